# baseline (speedup 1.0000x reference)
import functools

import jax
import jax.numpy as jnp
from jax import lax
from jax.experimental import pallas as pl
from jax.experimental.pallas import tpu as pltpu
from jax.experimental.pallas import tpu_sc as plsc

NC = 2
NS = 16
L = 16

_MESH = plsc.VectorSubcoreMesh(
    core_axis_name="c", subcore_axis_name="s", num_cores=NC, num_subcores=NS)


def _bins_per_tile(N, R):
    return ((N * R + NS - 1) // NS + 15) // 16 * 16


def _make_hist(N, E, R):
    BINS = _bins_per_tile(N, R)
    NKP = NS * BINS
    CH = 4000
    EPC = E // NC
    assert EPC % CH == 0 and BINS % 8 == 0

    def body(dst_hbm, typ_hbm, outa_hbm, outb_hbm, cnt_part,
             st_a0, st_b0, st_a1, st_b1, sd0, sd1):
        cid = lax.axis_index("c")
        sid = lax.axis_index("s")
        lo = sid * BINS
        ones = jnp.ones((L,), jnp.float32)
        sts = ((st_a0, st_b0, sd0), (st_a1, st_b1, sd1))

        def stage(ch, q):
            a, bb, sem = sts[q]
            off = cid * EPC + ch * CH
            pltpu.async_copy(dst_hbm.at[pl.ds(off, CH)], a, sem)
            pltpu.async_copy(typ_hbm.at[pl.ds(off, CH)], bb, sem)

        def stage_wait(ch, q):
            a, bb, sem = sts[q]
            off = cid * EPC + ch * CH
            pltpu.make_async_copy(dst_hbm.at[pl.ds(off, CH)], a, sem).wait()
            pltpu.make_async_copy(typ_hbm.at[pl.ds(off, CH)], bb, sem).wait()

        stage(0, 0)
        stage(1, 1)

        @plsc.parallel_loop(0, BINS // L, 1, unroll=4)
        def zero_body(i):
            cnt_part[pl.ds(i * L, L)] = jnp.zeros((L,), jnp.float32)

        def count_chunk(ch, _):
            q = lax.rem(ch, 2)

            def scan(a, bb):
                @plsc.parallel_loop(0, CH // L, 1, unroll=4)
                def count_vec(i):
                    dd = a[pl.ds(i * L, L)]
                    tt = bb[pl.ds(i * L, L)]
                    k = dd * R + tt - lo
                    m = (k >= 0) & (k < BINS)
                    kk = jnp.where(m, k, 0)
                    plsc.addupdate_scatter(cnt_part, [kk], ones, mask=m)

            def do(q_static):
                a, bb, _ = sts[q_static]
                stage_wait(ch, q_static)
                scan(a, bb)

                def prefetch():
                    stage(ch + 2, q_static)
                lax.cond(ch + 2 < EPC // CH, prefetch, lambda: None)

            lax.cond(q == 0, lambda: do(0), lambda: do(1))
            return 0
        lax.fori_loop(0, EPC // CH, count_chunk, 0)

        lax.cond(
            cid == 0,
            lambda: pltpu.sync_copy(cnt_part, outa_hbm.at[pl.ds(lo, BINS)]),
            lambda: pltpu.sync_copy(cnt_part, outb_hbm.at[pl.ds(lo, BINS)]))

    return pl.kernel(
        body,
        out_type=(jax.ShapeDtypeStruct((NKP,), jnp.float32),
                  jax.ShapeDtypeStruct((NKP,), jnp.float32)),
        mesh=_MESH,
        compiler_params=pltpu.CompilerParams(needs_layout_passes=False),
        scratch_types=[
            pltpu.VMEM((BINS,), jnp.float32),
            pltpu.VMEM((CH,), jnp.int32),
            pltpu.VMEM((CH,), jnp.int32),
            pltpu.VMEM((CH,), jnp.int32),
            pltpu.VMEM((CH,), jnp.int32),
            pltpu.SemaphoreType.DMA,
            pltpu.SemaphoreType.DMA,
        ],
    )


def _make_edgeprep(N, E, R):
    BINS = _bins_per_tile(N, R)
    NKP = NS * BINS
    EPT = E // (NC * NS)
    CH2 = 2000
    assert EPT % CH2 == 0

    def body(src_hbm, dst_hbm, typ_hbm, cnta_hbm, cntb_hbm, gidx_hbm, s_hbm,
             cnt_full, cbuf, st_s, st_d, st_t, st_g, st_sc):
        cid = lax.axis_index("c")
        sid = lax.axis_index("s")
        wid = cid * NS + sid

        pltpu.sync_copy(cnta_hbm, cnt_full)

        def merge_chunk(ch, _):
            pltpu.sync_copy(cntb_hbm.at[pl.ds(ch * BINS, BINS)], cbuf)

            @plsc.parallel_loop(0, BINS // L, 1, unroll=4)
            def merge_vec(i):
                off = ch * BINS + i * L
                cnt_full[pl.ds(off, L)] = (cnt_full[pl.ds(off, L)]
                                           + cbuf[pl.ds(i * L, L)])
            return 0
        lax.fori_loop(0, NS, merge_chunk, 0)

        base = wid * EPT

        def out_chunk(ch, _):
            off = base + ch * CH2
            pltpu.sync_copy(src_hbm.at[pl.ds(off, CH2)], st_s)
            pltpu.sync_copy(dst_hbm.at[pl.ds(off, CH2)], st_d)
            pltpu.sync_copy(typ_hbm.at[pl.ds(off, CH2)], st_t)

            @plsc.parallel_loop(0, CH2 // L, 1, unroll=2)
            def out_vec(i):
                ss = st_s[pl.ds(i * L, L)]
                dd = st_d[pl.ds(i * L, L)]
                tt = st_t[pl.ds(i * L, L)]
                st_g[pl.ds(i * L, L)] = ss * R + tt
                c = plsc.load_gather(cnt_full, [dd * R + tt])
                st_sc[pl.ds(i * L, L)] = 1.0 / jnp.maximum(c, 1.0)

            pltpu.sync_copy(st_g, gidx_hbm.at[pl.ds(off, CH2)])
            pltpu.sync_copy(st_sc, s_hbm.at[pl.ds(off, CH2)])
            return 0
        lax.fori_loop(0, EPT // CH2, out_chunk, 0)

    return pl.kernel(
        body,
        out_type=(jax.ShapeDtypeStruct((E,), jnp.int32),
                  jax.ShapeDtypeStruct((E,), jnp.float32)),
        mesh=_MESH,
        compiler_params=pltpu.CompilerParams(needs_layout_passes=False),
        scratch_types=[
            pltpu.VMEM((NKP,), jnp.float32),
            pltpu.VMEM((BINS,), jnp.float32),
            pltpu.VMEM((CH2,), jnp.int32),
            pltpu.VMEM((CH2,), jnp.int32),
            pltpu.VMEM((CH2,), jnp.int32),
            pltpu.VMEM((CH2,), jnp.int32),
            pltpu.VMEM((CH2,), jnp.float32),
        ],
    )


_KB = 125
_CHR = 16
_NBUF = 4


def _make_aggregate(N, E, R, D):
    K = _KB
    H = D // 2
    EPT = E // NS
    NB = EPT // K
    NCH = NB // _CHR
    EC = _CHR * K
    RPT = -(-(N // NS) // 8) * 8
    NP = NS * RPT
    assert EPT % K == 0 and NB % _CHR == 0 and _CHR % _NBUF == 0
    NJ = H // L
    NZ = RPT // K
    ZREM = RPT - NZ * K

    def body(ya_hbm, yb_hbm, gidx_hbm, dst_hbm, s_hbm, out_hbm,
             acc_sh, gidx_c0, gidx_c1, dst_c0, dst_c1, s_c0, s_c1,
             rows0, rows1, rows2, rows3,
             st0, st1, sg0, sg1, sg2, sg3, ss0, ss1, ss2, ss3):
        cid = lax.axis_index("c")
        sid = lax.axis_index("s")
        st = (st0, st1)
        gi = (gidx_c0, gidx_c1)
        di = (dst_c0, dst_c1)
        sv = (s_c0, s_c1)
        rows = (rows0, rows1, rows2, rows3)
        sg = (sg0, sg1, sg2, sg3)
        ss = (ss0, ss1, ss2, ss3)

        def stage(ch, q):
            pltpu.async_copy(gidx_hbm.at[sid, pl.ds(ch * _CHR, _CHR)], gi[q], st[q])
            pltpu.async_copy(dst_hbm.at[sid, pl.ds(ch * _CHR, _CHR)], di[q], st[q])
            pltpu.async_copy(s_hbm.at[pl.ds(sid * EPT + ch * EC, EC)],
                             sv[q].at[pl.ds(0, EC)], st[q])

        def stage_wait(ch, q):
            pltpu.make_async_copy(gidx_hbm.at[sid, pl.ds(ch * _CHR, _CHR)],
                                  gi[q], st[q]).wait()
            pltpu.make_async_copy(dst_hbm.at[sid, pl.ds(ch * _CHR, _CHR)],
                                  di[q], st[q]).wait()
            pltpu.make_async_copy(s_hbm.at[pl.ds(sid * EPT + ch * EC, EC)],
                                  sv[q].at[pl.ds(0, EC)], st[q]).wait()

        stage(0, 0)
        if NCH > 1:
            stage(1, 1)

        def zfill(i, _):
            for j in range(NJ):
                rows0[i, pl.ds(j * L, L)] = jnp.zeros((L,), jnp.float32)
            return 0
        lax.fori_loop(0, K, zfill, 0)

        def zcopy(j, _):
            pltpu.sync_copy(rows0, acc_sh.at[pl.ds(sid * RPT + j * K, K)])
            return 0
        lax.fori_loop(0, NZ, zcopy, 0)
        if ZREM:
            pltpu.sync_copy(rows0.at[pl.ds(0, ZREM)],
                            acc_sh.at[pl.ds(sid * RPT + NZ * K, ZREM)])
        plsc.subcore_barrier()

        def gather(q, b, r):
            def ga():
                pltpu.async_copy(ya_hbm.at[gi[q].at[b]], rows[r], sg[r])

            def gb():
                pltpu.async_copy(yb_hbm.at[gi[q].at[b]], rows[r], sg[r])
            lax.cond(cid == 0, ga, gb)

        def gather_wait(q, b, r):
            def ga():
                pltpu.make_async_copy(ya_hbm.at[gi[q].at[b]], rows[r], sg[r]).wait()

            def gb():
                pltpu.make_async_copy(yb_hbm.at[gi[q].at[b]], rows[r], sg[r]).wait()
            lax.cond(cid == 0, ga, gb)

        def scatter(q, b, r):
            pltpu.async_copy(rows[r], acc_sh.at[di[q].at[b]], ss[r], add=True)

        def scatter_wait(q, b, r):
            pltpu.make_async_copy(rows[r], acc_sh.at[di[q].at[b]], ss[r]).wait()

        def scale(q, b, r):
            buf = rows[r]

            def scale_e(e, _):
                se = sv[q][pl.ds(b * K + e, L)][0]
                for j in range(NJ):
                    buf[e, pl.ds(j * L, L)] = buf[e, pl.ds(j * L, L)] * se
                return 0
            lax.fori_loop(0, K, scale_e, 0)

        def quad(q, b0, prefetch):
            for t in range(_NBUF):
                gather_wait(q, b0 + t, t)
                scale(q, b0 + t, t)
                scatter(q, b0 + t, t)
            for t in range(_NBUF):
                scatter_wait(q, b0 + t, t)
                if prefetch:
                    gather(q, b0 + t + _NBUF, t)
            return 0

        NQ = _CHR // _NBUF
        stage_wait(0, 0)
        for t in range(_NBUF):
            gather(0, t, t)
        for ch in range(NCH):
            q = ch % 2
            lax.fori_loop(0, NQ - 1, lambda p, c: quad(q, _NBUF * p, True), 0)
            quad(q, _CHR - _NBUF, False)
            if ch + 1 < NCH:
                stage_wait(ch + 1, 1 - q)
                for t in range(_NBUF):
                    gather(1 - q, t, t)
            if ch + 2 < NCH:
                stage(ch + 2, q)
        plsc.subcore_barrier()

        pltpu.sync_copy(acc_sh.at[pl.ds(sid * RPT, RPT)],
                        out_hbm.at[cid, pl.ds(sid * RPT, RPT)])

    return pl.kernel(
        body,
        out_type=jax.ShapeDtypeStruct((NC, NP, H), jnp.float32),
        mesh=_MESH,
        compiler_params=pltpu.CompilerParams(needs_layout_passes=False,
                                             use_tc_tiling_on_sc=False),
        scratch_types=[
            pltpu.VMEM_SHARED((NP, H), jnp.float32),
            pltpu.VMEM((_CHR, K), jnp.int32),
            pltpu.VMEM((_CHR, K), jnp.int32),
            pltpu.VMEM((_CHR, K), jnp.int32),
            pltpu.VMEM((_CHR, K), jnp.int32),
            pltpu.VMEM((EC + L,), jnp.float32),
            pltpu.VMEM((EC + L,), jnp.float32),
            pltpu.VMEM((K, H), jnp.float32),
            pltpu.VMEM((K, H), jnp.float32),
            pltpu.VMEM((K, H), jnp.float32),
            pltpu.VMEM((K, H), jnp.float32),
            pltpu.SemaphoreType.DMA,
            pltpu.SemaphoreType.DMA,
            pltpu.SemaphoreType.DMA,
            pltpu.SemaphoreType.DMA,
            pltpu.SemaphoreType.DMA,
            pltpu.SemaphoreType.DMA,
            pltpu.SemaphoreType.DMA,
            pltpu.SemaphoreType.DMA,
            pltpu.SemaphoreType.DMA,
            pltpu.SemaphoreType.DMA,
        ],
    )


def _tc_first(x, wc, BM=400):
    N, D = x.shape
    CT = wc.shape[1]
    RH = (CT - D) // 2

    def body(x_ref, wc_ref, ya_ref, yb_ref, z_ref):
        y = jnp.dot(x_ref[...], wc_ref[...], preferred_element_type=jnp.float32)
        ya_ref[...] = y[:, :RH]
        yb_ref[...] = y[:, RH:2 * RH]
        z_ref[...] = y[:, 2 * RH:]

    return pl.pallas_call(
        body,
        grid=(N // BM,),
        in_specs=[
            pl.BlockSpec((BM, D), lambda i: (i, 0)),
            pl.BlockSpec((D, CT), lambda i: (0, 0)),
        ],
        out_specs=[
            pl.BlockSpec((BM, RH), lambda i: (i, 0)),
            pl.BlockSpec((BM, RH), lambda i: (i, 0)),
            pl.BlockSpec((BM, D), lambda i: (i, 0)),
        ],
        out_shape=[
            jax.ShapeDtypeStruct((N, RH), jnp.float32),
            jax.ShapeDtypeStruct((N, RH), jnp.float32),
            jax.ShapeDtypeStruct((N, D), jnp.float32),
        ],
    )(x, wc)


def _tc_mid(z, a0, a1, b, wc, BM=400):
    N, D = z.shape
    H = D // 2
    CT = wc.shape[1]
    RH = (CT - D) // 2

    def body(z_ref, a0_ref, a1_ref, b_ref, wc_ref, ya_ref, yb_ref, zo_ref):
        agg = jnp.concatenate([a0_ref[...], a1_ref[...]], axis=1)
        h = jnp.maximum(z_ref[...] + agg + b_ref[...], 0.0)
        y = jnp.dot(h, wc_ref[...], preferred_element_type=jnp.float32)
        ya_ref[...] = y[:, :RH]
        yb_ref[...] = y[:, RH:2 * RH]
        zo_ref[...] = y[:, 2 * RH:]

    return pl.pallas_call(
        body,
        grid=(N // BM,),
        in_specs=[
            pl.BlockSpec((BM, D), lambda i: (i, 0)),
            pl.BlockSpec((BM, H), lambda i: (i, 0)),
            pl.BlockSpec((BM, H), lambda i: (i, 0)),
            pl.BlockSpec((1, D), lambda i: (0, 0)),
            pl.BlockSpec((D, CT), lambda i: (0, 0)),
        ],
        out_specs=[
            pl.BlockSpec((BM, RH), lambda i: (i, 0)),
            pl.BlockSpec((BM, RH), lambda i: (i, 0)),
            pl.BlockSpec((BM, D), lambda i: (i, 0)),
        ],
        out_shape=[
            jax.ShapeDtypeStruct((N, RH), jnp.float32),
            jax.ShapeDtypeStruct((N, RH), jnp.float32),
            jax.ShapeDtypeStruct((N, D), jnp.float32),
        ],
    )(z, a0, a1, b, wc)


def _tc_last(z, a0, a1, b, BM=400):
    N, D = z.shape
    H = D // 2

    def body(z_ref, a0_ref, a1_ref, b_ref, h_ref):
        agg = jnp.concatenate([a0_ref[...], a1_ref[...]], axis=1)
        h_ref[...] = jnp.maximum(z_ref[...] + agg + b_ref[...], 0.0)

    return pl.pallas_call(
        body,
        grid=(N // BM,),
        in_specs=[
            pl.BlockSpec((BM, D), lambda i: (i, 0)),
            pl.BlockSpec((BM, H), lambda i: (i, 0)),
            pl.BlockSpec((BM, H), lambda i: (i, 0)),
            pl.BlockSpec((1, D), lambda i: (0, 0)),
        ],
        out_specs=pl.BlockSpec((BM, D), lambda i: (i, 0)),
        out_shape=jax.ShapeDtypeStruct((N, D), jnp.float32),
    )(z, a0, a1, b)


def kernel(x, edge_index, edge_type, w1, r1, b1, w2, r2, b2, w3, r3, b3):
    N, D = x.shape
    E = edge_type.shape[0]
    R = w1.shape[0]
    H = D // 2

    src = edge_index[0].astype(jnp.int32)
    dst = edge_index[1].astype(jnp.int32)
    typ = edge_type.astype(jnp.int32)

    cnta, cntb = _make_hist(N, E, R)(dst, typ)
    gidx, s = _make_edgeprep(N, E, R)(src, dst, typ, cnta, cntb)
    agg = _make_aggregate(N, E, R, D)
    gidx2 = gidx.reshape(NS, E // (NS * _KB), _KB)
    dst2 = dst.reshape(NS, E // (NS * _KB), _KB)

    def wcat(w, root):
        wt = jnp.transpose(w, (1, 0, 2))
        wl = wt[:, :, :H].reshape(D, R * H)
        wr_ = wt[:, :, H:].reshape(D, R * H)
        return jnp.concatenate([wl, wr_, root], axis=1)

    y1a, y1b, z1 = _tc_first(x, wcat(w1, r1))
    acc1 = agg(y1a.reshape(N * R, H), y1b.reshape(N * R, H), gidx2, dst2, s)
    y2a, y2b, z2 = _tc_mid(z1, acc1[0], acc1[1], b1.reshape(1, D), wcat(w2, r2))
    acc2 = agg(y2a.reshape(N * R, H), y2b.reshape(N * R, H), gidx2, dst2, s)
    y3a, y3b, z3 = _tc_mid(z2, acc2[0], acc2[1], b2.reshape(1, D), wcat(w3, r3))
    acc3 = agg(y3a.reshape(N * R, H), y3b.reshape(N * R, H), gidx2, dst2, s)
    return _tc_last(z3, acc3[0], acc3[1], b3.reshape(1, D))


# speedup vs baseline: 1.4031x; 1.4031x over previous
import functools

import jax
import jax.numpy as jnp
from jax import lax
from jax.experimental import pallas as pl
from jax.experimental.pallas import tpu as pltpu
from jax.experimental.pallas import tpu_sc as plsc

NC = 2
NS = 16
L = 16

_MESH = plsc.VectorSubcoreMesh(
    core_axis_name="c", subcore_axis_name="s", num_cores=NC, num_subcores=NS)


def _bins_per_tile(N, R):
    return ((N * R + NS - 1) // NS + 15) // 16 * 16


def _make_hist(N, E, R):
    BINS = _bins_per_tile(N, R)
    NKP = NS * BINS
    CH = 4000
    EPC = E // NC
    assert EPC % CH == 0 and BINS % 8 == 0

    def body(dst_hbm, typ_hbm, outa_hbm, outb_hbm, cnt_part,
             st_a0, st_b0, st_a1, st_b1, sd0, sd1):
        cid = lax.axis_index("c")
        sid = lax.axis_index("s")
        lo = sid * BINS
        ones = jnp.ones((L,), jnp.float32)
        sts = ((st_a0, st_b0, sd0), (st_a1, st_b1, sd1))

        def stage(ch, q):
            a, bb, sem = sts[q]
            off = cid * EPC + ch * CH
            pltpu.async_copy(dst_hbm.at[pl.ds(off, CH)], a, sem)
            pltpu.async_copy(typ_hbm.at[pl.ds(off, CH)], bb, sem)

        def stage_wait(ch, q):
            a, bb, sem = sts[q]
            off = cid * EPC + ch * CH
            pltpu.make_async_copy(dst_hbm.at[pl.ds(off, CH)], a, sem).wait()
            pltpu.make_async_copy(typ_hbm.at[pl.ds(off, CH)], bb, sem).wait()

        stage(0, 0)
        stage(1, 1)

        @plsc.parallel_loop(0, BINS // L, 1, unroll=4)
        def zero_body(i):
            cnt_part[pl.ds(i * L, L)] = jnp.zeros((L,), jnp.float32)

        def count_chunk(ch, _):
            q = lax.rem(ch, 2)

            def scan(a, bb):
                @plsc.parallel_loop(0, CH // L, 1, unroll=4)
                def count_vec(i):
                    dd = a[pl.ds(i * L, L)]
                    tt = bb[pl.ds(i * L, L)]
                    k = dd * R + tt - lo
                    m = (k >= 0) & (k < BINS)
                    kk = jnp.where(m, k, 0)
                    plsc.addupdate_scatter(cnt_part, [kk], ones, mask=m)

            def do(q_static):
                a, bb, _ = sts[q_static]
                stage_wait(ch, q_static)
                scan(a, bb)

                def prefetch():
                    stage(ch + 2, q_static)
                lax.cond(ch + 2 < EPC // CH, prefetch, lambda: None)

            lax.cond(q == 0, lambda: do(0), lambda: do(1))
            return 0
        lax.fori_loop(0, EPC // CH, count_chunk, 0)

        lax.cond(
            cid == 0,
            lambda: pltpu.sync_copy(cnt_part, outa_hbm.at[pl.ds(lo, BINS)]),
            lambda: pltpu.sync_copy(cnt_part, outb_hbm.at[pl.ds(lo, BINS)]))

    return pl.kernel(
        body,
        out_type=(jax.ShapeDtypeStruct((NKP,), jnp.float32),
                  jax.ShapeDtypeStruct((NKP,), jnp.float32)),
        mesh=_MESH,
        compiler_params=pltpu.CompilerParams(needs_layout_passes=False),
        scratch_types=[
            pltpu.VMEM((BINS,), jnp.float32),
            pltpu.VMEM((CH,), jnp.int32),
            pltpu.VMEM((CH,), jnp.int32),
            pltpu.VMEM((CH,), jnp.int32),
            pltpu.VMEM((CH,), jnp.int32),
            pltpu.SemaphoreType.DMA,
            pltpu.SemaphoreType.DMA,
        ],
    )


def _make_edgeprep(N, E, R):
    BINS = _bins_per_tile(N, R)
    NKP = NS * BINS
    EPT = E // (NC * NS)
    CH2 = 2000
    assert EPT % CH2 == 0

    def body(src_hbm, dst_hbm, typ_hbm, cnta_hbm, cntb_hbm, gidx_hbm, s_hbm,
             cnt_full, cbuf, st_s, st_d, st_t, st_g, st_sc):
        cid = lax.axis_index("c")
        sid = lax.axis_index("s")
        wid = cid * NS + sid

        pltpu.sync_copy(cnta_hbm, cnt_full)

        def merge_chunk(ch, _):
            pltpu.sync_copy(cntb_hbm.at[pl.ds(ch * BINS, BINS)], cbuf)

            @plsc.parallel_loop(0, BINS // L, 1, unroll=4)
            def merge_vec(i):
                off = ch * BINS + i * L
                cnt_full[pl.ds(off, L)] = (cnt_full[pl.ds(off, L)]
                                           + cbuf[pl.ds(i * L, L)])
            return 0
        lax.fori_loop(0, NS, merge_chunk, 0)

        base = wid * EPT

        def out_chunk(ch, _):
            off = base + ch * CH2
            pltpu.sync_copy(src_hbm.at[pl.ds(off, CH2)], st_s)
            pltpu.sync_copy(dst_hbm.at[pl.ds(off, CH2)], st_d)
            pltpu.sync_copy(typ_hbm.at[pl.ds(off, CH2)], st_t)

            @plsc.parallel_loop(0, CH2 // L, 1, unroll=2)
            def out_vec(i):
                ss = st_s[pl.ds(i * L, L)]
                dd = st_d[pl.ds(i * L, L)]
                tt = st_t[pl.ds(i * L, L)]
                st_g[pl.ds(i * L, L)] = ss * R + tt
                c = plsc.load_gather(cnt_full, [dd * R + tt])
                st_sc[pl.ds(i * L, L)] = 1.0 / jnp.maximum(c, 1.0)

            pltpu.sync_copy(st_g, gidx_hbm.at[pl.ds(off, CH2)])
            pltpu.sync_copy(st_sc, s_hbm.at[pl.ds(off, CH2)])
            return 0
        lax.fori_loop(0, EPT // CH2, out_chunk, 0)

    return pl.kernel(
        body,
        out_type=(jax.ShapeDtypeStruct((E,), jnp.int32),
                  jax.ShapeDtypeStruct((E,), jnp.float32)),
        mesh=_MESH,
        compiler_params=pltpu.CompilerParams(needs_layout_passes=False),
        scratch_types=[
            pltpu.VMEM((NKP,), jnp.float32),
            pltpu.VMEM((BINS,), jnp.float32),
            pltpu.VMEM((CH2,), jnp.int32),
            pltpu.VMEM((CH2,), jnp.int32),
            pltpu.VMEM((CH2,), jnp.int32),
            pltpu.VMEM((CH2,), jnp.int32),
            pltpu.VMEM((CH2,), jnp.float32),
        ],
    )


_KB = 125
_CHR = 16


def _make_aggregate(N, E, R, D):
    K = _KB
    NW = NC * NS
    EPT = E // NW
    NB = EPT // K
    NCH = NB // _CHR
    EC = _CHR * K
    RPT = -(-(N // NS) // 8) * 8
    NP = NS * RPT
    assert EPT % K == 0 and NB % _CHR == 0 and _CHR % 2 == 0
    NJ = D // L
    NZ = RPT // K
    ZREM = RPT - NZ * K

    def body(y_hbm, gidx_hbm, dst_hbm, s_hbm, out_hbm,
             acc_sh, gidx_c0, gidx_c1, dst_c0, dst_c1, s_c0, s_c1,
             rows0, rows1, st0, st1, sg0, sg1, ss0, ss1):
        cid = lax.axis_index("c")
        sid = lax.axis_index("s")
        wid = cid * NS + sid
        st = (st0, st1)
        gi = (gidx_c0, gidx_c1)
        di = (dst_c0, dst_c1)
        sv = (s_c0, s_c1)

        def stage(ch, q):
            pltpu.async_copy(gidx_hbm.at[wid, pl.ds(ch * _CHR, _CHR)], gi[q], st[q])
            pltpu.async_copy(dst_hbm.at[wid, pl.ds(ch * _CHR, _CHR)], di[q], st[q])
            pltpu.async_copy(s_hbm.at[pl.ds(wid * EPT + ch * EC, EC)],
                             sv[q].at[pl.ds(0, EC)], st[q])

        def stage_wait(ch, q):
            pltpu.make_async_copy(gidx_hbm.at[wid, pl.ds(ch * _CHR, _CHR)],
                                  gi[q], st[q]).wait()
            pltpu.make_async_copy(dst_hbm.at[wid, pl.ds(ch * _CHR, _CHR)],
                                  di[q], st[q]).wait()
            pltpu.make_async_copy(s_hbm.at[pl.ds(wid * EPT + ch * EC, EC)],
                                  sv[q].at[pl.ds(0, EC)], st[q]).wait()

        stage(0, 0)
        if NCH > 1:
            stage(1, 1)

        def zfill(i, _):
            for j in range(NJ):
                rows0[i, pl.ds(j * L, L)] = jnp.zeros((L,), jnp.float32)
            return 0
        lax.fori_loop(0, K, zfill, 0)

        def zcopy(j, _):
            pltpu.sync_copy(rows0, acc_sh.at[pl.ds(sid * RPT + j * K, K)])
            return 0
        lax.fori_loop(0, NZ, zcopy, 0)
        if ZREM:
            pltpu.sync_copy(rows0.at[pl.ds(0, ZREM)],
                            acc_sh.at[pl.ds(sid * RPT + NZ * K, ZREM)])
        plsc.subcore_barrier()

        def gather(q, b, buf, sem):
            pltpu.async_copy(y_hbm.at[gi[q].at[b]], buf, sem)

        def gather_wait(q, b, buf, sem):
            pltpu.make_async_copy(y_hbm.at[gi[q].at[b]], buf, sem).wait()

        def scatter(q, b, buf, sem):
            pltpu.async_copy(buf, acc_sh.at[di[q].at[b]], sem, add=True)

        def scatter_wait(q, b, buf, sem):
            pltpu.make_async_copy(buf, acc_sh.at[di[q].at[b]], sem).wait()

        def scale(q, b, buf):
            def scale_e(i, _):
                e = 2 * i
                se0 = sv[q][pl.ds(b * K + e, L)][0]
                se1 = sv[q][pl.ds(b * K + e + 1, L)][0]
                for j in range(NJ):
                    buf[e, pl.ds(j * L, L)] = buf[e, pl.ds(j * L, L)] * se0
                for j in range(NJ):
                    buf[e + 1, pl.ds(j * L, L)] = buf[e + 1, pl.ds(j * L, L)] * se1
                return 0
            lax.fori_loop(0, K // 2, scale_e, 0)
            if K % 2:
                se = sv[q][pl.ds(b * K + K - 1, L)][0]
                for j in range(NJ):
                    buf[K - 1, pl.ds(j * L, L)] = buf[K - 1, pl.ds(j * L, L)] * se

        def pair(q, b0, prefetch):
            b1 = b0 + 1
            gather_wait(q, b0, rows0, sg0)
            scale(q, b0, rows0)
            scatter(q, b0, rows0, ss0)
            gather_wait(q, b1, rows1, sg1)
            scale(q, b1, rows1)
            scatter(q, b1, rows1, ss1)
            scatter_wait(q, b0, rows0, ss0)
            if prefetch:
                gather(q, b0 + 2, rows0, sg0)
            scatter_wait(q, b1, rows1, ss1)
            if prefetch:
                gather(q, b1 + 2, rows1, sg1)
            return 0

        stage_wait(0, 0)
        gather(0, 0, rows0, sg0)
        gather(0, 1, rows1, sg1)
        for ch in range(NCH):
            q = ch % 2
            lax.fori_loop(0, _CHR // 2 - 1, lambda p, c: pair(q, 2 * p, True), 0)
            pair(q, _CHR - 2, False)
            if ch + 1 < NCH:
                stage_wait(ch + 1, 1 - q)
                gather(1 - q, 0, rows0, sg0)
                gather(1 - q, 1, rows1, sg1)
            if ch + 2 < NCH:
                stage(ch + 2, q)
        plsc.subcore_barrier()

        pltpu.sync_copy(acc_sh.at[pl.ds(sid * RPT, RPT)],
                        out_hbm.at[cid, pl.ds(sid * RPT, RPT)])

    return pl.kernel(
        body,
        out_type=jax.ShapeDtypeStruct((NC, NP, D), jnp.float32),
        mesh=_MESH,
        compiler_params=pltpu.CompilerParams(needs_layout_passes=False),
        scratch_types=[
            pltpu.VMEM_SHARED((NP, D), jnp.float32),
            pltpu.VMEM((_CHR, K), jnp.int32),
            pltpu.VMEM((_CHR, K), jnp.int32),
            pltpu.VMEM((_CHR, K), jnp.int32),
            pltpu.VMEM((_CHR, K), jnp.int32),
            pltpu.VMEM((EC + L,), jnp.float32),
            pltpu.VMEM((EC + L,), jnp.float32),
            pltpu.VMEM((K, D), jnp.float32),
            pltpu.VMEM((K, D), jnp.float32),
            pltpu.SemaphoreType.DMA,
            pltpu.SemaphoreType.DMA,
            pltpu.SemaphoreType.DMA,
            pltpu.SemaphoreType.DMA,
            pltpu.SemaphoreType.DMA,
            pltpu.SemaphoreType.DMA,
        ],
    )


def _tc_first(x, wc, wr, BM=400):
    N, D = x.shape
    RD = wc.shape[1]

    def body(x_ref, wc_ref, wr_ref, y_ref, z_ref):
        xb = x_ref[...]
        y_ref[...] = jnp.dot(xb, wc_ref[...], preferred_element_type=jnp.float32)
        z_ref[...] = jnp.dot(xb, wr_ref[...], preferred_element_type=jnp.float32)

    return pl.pallas_call(
        body,
        grid=(N // BM,),
        in_specs=[
            pl.BlockSpec((BM, D), lambda i: (i, 0)),
            pl.BlockSpec((D, RD), lambda i: (0, 0)),
            pl.BlockSpec((D, D), lambda i: (0, 0)),
        ],
        out_specs=[
            pl.BlockSpec((BM, RD), lambda i: (i, 0)),
            pl.BlockSpec((BM, D), lambda i: (i, 0)),
        ],
        out_shape=[
            jax.ShapeDtypeStruct((N, RD), jnp.float32),
            jax.ShapeDtypeStruct((N, D), jnp.float32),
        ],
    )(x, wc, wr)


def _tc_mid(z, a0, a1, b, wc, wr, BM=400):
    N, D = z.shape
    RD = wc.shape[1]

    def body(z_ref, a0_ref, a1_ref, b_ref, wc_ref, wr_ref, y_ref, zo_ref):
        h = jnp.maximum(z_ref[...] + a0_ref[...] + a1_ref[...] + b_ref[...], 0.0)
        y_ref[...] = jnp.dot(h, wc_ref[...], preferred_element_type=jnp.float32)
        zo_ref[...] = jnp.dot(h, wr_ref[...], preferred_element_type=jnp.float32)

    return pl.pallas_call(
        body,
        grid=(N // BM,),
        in_specs=[
            pl.BlockSpec((BM, D), lambda i: (i, 0)),
            pl.BlockSpec((BM, D), lambda i: (i, 0)),
            pl.BlockSpec((BM, D), lambda i: (i, 0)),
            pl.BlockSpec((1, D), lambda i: (0, 0)),
            pl.BlockSpec((D, RD), lambda i: (0, 0)),
            pl.BlockSpec((D, D), lambda i: (0, 0)),
        ],
        out_specs=[
            pl.BlockSpec((BM, RD), lambda i: (i, 0)),
            pl.BlockSpec((BM, D), lambda i: (i, 0)),
        ],
        out_shape=[
            jax.ShapeDtypeStruct((N, RD), jnp.float32),
            jax.ShapeDtypeStruct((N, D), jnp.float32),
        ],
    )(z, a0, a1, b, wc, wr)


def _tc_last(z, a0, a1, b, BM=400):
    N, D = z.shape

    def body(z_ref, a0_ref, a1_ref, b_ref, h_ref):
        h_ref[...] = jnp.maximum(
            z_ref[...] + a0_ref[...] + a1_ref[...] + b_ref[...], 0.0)

    return pl.pallas_call(
        body,
        grid=(N // BM,),
        in_specs=[
            pl.BlockSpec((BM, D), lambda i: (i, 0)),
            pl.BlockSpec((BM, D), lambda i: (i, 0)),
            pl.BlockSpec((BM, D), lambda i: (i, 0)),
            pl.BlockSpec((1, D), lambda i: (0, 0)),
        ],
        out_specs=pl.BlockSpec((BM, D), lambda i: (i, 0)),
        out_shape=jax.ShapeDtypeStruct((N, D), jnp.float32),
    )(z, a0, a1, b)


def kernel(x, edge_index, edge_type, w1, r1, b1, w2, r2, b2, w3, r3, b3):
    N, D = x.shape
    E = edge_type.shape[0]
    R = w1.shape[0]

    src = edge_index[0].astype(jnp.int32)
    dst = edge_index[1].astype(jnp.int32)
    typ = edge_type.astype(jnp.int32)

    cnta, cntb = _make_hist(N, E, R)(dst, typ)
    gidx, s = _make_edgeprep(N, E, R)(src, dst, typ, cnta, cntb)
    agg = _make_aggregate(N, E, R, D)
    NW = NC * NS
    gidx2 = gidx.reshape(NW, E // (NW * _KB), _KB)
    dst2 = dst.reshape(NW, E // (NW * _KB), _KB)

    def wcat(w):
        return jnp.transpose(w, (1, 0, 2)).reshape(D, R * D)

    y1, z1 = _tc_first(x, wcat(w1), r1)
    acc1 = agg(y1.reshape(N * R, D), gidx2, dst2, s)
    y2, z2 = _tc_mid(z1, acc1[0], acc1[1], b1.reshape(1, D), wcat(w2), r2)
    acc2 = agg(y2.reshape(N * R, D), gidx2, dst2, s)
    y3, z3 = _tc_mid(z2, acc2[0], acc2[1], b2.reshape(1, D), wcat(w3), r3)
    acc3 = agg(y3.reshape(N * R, D), gidx2, dst2, s)
    return _tc_last(z3, acc3[0], acc3[1], b3.reshape(1, D))


# speedup vs baseline: 1.4569x; 1.0383x over previous
import functools

import jax
import jax.numpy as jnp
from jax import lax
from jax.experimental import pallas as pl
from jax.experimental.pallas import tpu as pltpu
from jax.experimental.pallas import tpu_sc as plsc

NC = 2
NS = 16
L = 16

_MESH = plsc.VectorSubcoreMesh(
    core_axis_name="c", subcore_axis_name="s", num_cores=NC, num_subcores=NS)


def _bins_per_tile(N, R):
    return ((N * R + NS - 1) // NS + 15) // 16 * 16


def _make_hist(N, E, R):
    BINS = _bins_per_tile(N, R)
    NKP = NS * BINS
    CH = 4000
    EPC = E // NC
    assert EPC % CH == 0 and BINS % 8 == 0

    def body(dst_hbm, typ_hbm, outa_hbm, outb_hbm, cnt_part,
             st_a0, st_b0, st_a1, st_b1, sd0, sd1):
        cid = lax.axis_index("c")
        sid = lax.axis_index("s")
        lo = sid * BINS
        ones = jnp.ones((L,), jnp.float32)
        sts = ((st_a0, st_b0, sd0), (st_a1, st_b1, sd1))

        def stage(ch, q):
            a, bb, sem = sts[q]
            off = cid * EPC + ch * CH
            pltpu.async_copy(dst_hbm.at[pl.ds(off, CH)], a, sem)
            pltpu.async_copy(typ_hbm.at[pl.ds(off, CH)], bb, sem)

        def stage_wait(ch, q):
            a, bb, sem = sts[q]
            off = cid * EPC + ch * CH
            pltpu.make_async_copy(dst_hbm.at[pl.ds(off, CH)], a, sem).wait()
            pltpu.make_async_copy(typ_hbm.at[pl.ds(off, CH)], bb, sem).wait()

        stage(0, 0)
        stage(1, 1)

        @plsc.parallel_loop(0, BINS // L, 1, unroll=4)
        def zero_body(i):
            cnt_part[pl.ds(i * L, L)] = jnp.zeros((L,), jnp.float32)

        def count_chunk(ch, _):
            q = lax.rem(ch, 2)

            def scan(a, bb):
                @plsc.parallel_loop(0, CH // L, 1, unroll=4)
                def count_vec(i):
                    dd = a[pl.ds(i * L, L)]
                    tt = bb[pl.ds(i * L, L)]
                    k = dd * R + tt - lo
                    m = (k >= 0) & (k < BINS)
                    kk = jnp.where(m, k, 0)
                    plsc.addupdate_scatter(cnt_part, [kk], ones, mask=m)

            def do(q_static):
                a, bb, _ = sts[q_static]
                stage_wait(ch, q_static)
                scan(a, bb)

                def prefetch():
                    stage(ch + 2, q_static)
                lax.cond(ch + 2 < EPC // CH, prefetch, lambda: None)

            lax.cond(q == 0, lambda: do(0), lambda: do(1))
            return 0
        lax.fori_loop(0, EPC // CH, count_chunk, 0)

        lax.cond(
            cid == 0,
            lambda: pltpu.sync_copy(cnt_part, outa_hbm.at[pl.ds(lo, BINS)]),
            lambda: pltpu.sync_copy(cnt_part, outb_hbm.at[pl.ds(lo, BINS)]))

    return pl.kernel(
        body,
        out_type=(jax.ShapeDtypeStruct((NKP,), jnp.float32),
                  jax.ShapeDtypeStruct((NKP,), jnp.float32)),
        mesh=_MESH,
        compiler_params=pltpu.CompilerParams(needs_layout_passes=False),
        scratch_types=[
            pltpu.VMEM((BINS,), jnp.float32),
            pltpu.VMEM((CH,), jnp.int32),
            pltpu.VMEM((CH,), jnp.int32),
            pltpu.VMEM((CH,), jnp.int32),
            pltpu.VMEM((CH,), jnp.int32),
            pltpu.SemaphoreType.DMA,
            pltpu.SemaphoreType.DMA,
        ],
    )


def _make_edgeprep(N, E, R):
    BINS = _bins_per_tile(N, R)
    NKP = NS * BINS
    EPT = E // (NC * NS)
    CH2 = 2000
    assert EPT % CH2 == 0

    def body(src_hbm, dst_hbm, typ_hbm, cnta_hbm, cntb_hbm, gidx_hbm, s_hbm,
             cnt_full, cbuf, st_s, st_d, st_t, st_g, st_sc):
        cid = lax.axis_index("c")
        sid = lax.axis_index("s")
        wid = cid * NS + sid

        pltpu.sync_copy(cnta_hbm, cnt_full)

        def merge_chunk(ch, _):
            pltpu.sync_copy(cntb_hbm.at[pl.ds(ch * BINS, BINS)], cbuf)

            @plsc.parallel_loop(0, BINS // L, 1, unroll=4)
            def merge_vec(i):
                off = ch * BINS + i * L
                cnt_full[pl.ds(off, L)] = (cnt_full[pl.ds(off, L)]
                                           + cbuf[pl.ds(i * L, L)])
            return 0
        lax.fori_loop(0, NS, merge_chunk, 0)

        base = wid * EPT

        def out_chunk(ch, _):
            off = base + ch * CH2
            pltpu.sync_copy(src_hbm.at[pl.ds(off, CH2)], st_s)
            pltpu.sync_copy(dst_hbm.at[pl.ds(off, CH2)], st_d)
            pltpu.sync_copy(typ_hbm.at[pl.ds(off, CH2)], st_t)

            @plsc.parallel_loop(0, CH2 // L, 1, unroll=2)
            def out_vec(i):
                ss = st_s[pl.ds(i * L, L)]
                dd = st_d[pl.ds(i * L, L)]
                tt = st_t[pl.ds(i * L, L)]
                st_g[pl.ds(i * L, L)] = ss * R + tt
                c = plsc.load_gather(cnt_full, [dd * R + tt])
                st_sc[pl.ds(i * L, L)] = 1.0 / jnp.maximum(c, 1.0)

            pltpu.sync_copy(st_g, gidx_hbm.at[pl.ds(off, CH2)])
            pltpu.sync_copy(st_sc, s_hbm.at[pl.ds(off, CH2)])
            return 0
        lax.fori_loop(0, EPT // CH2, out_chunk, 0)

    return pl.kernel(
        body,
        out_type=(jax.ShapeDtypeStruct((E,), jnp.int32),
                  jax.ShapeDtypeStruct((E,), jnp.float32)),
        mesh=_MESH,
        compiler_params=pltpu.CompilerParams(needs_layout_passes=False),
        scratch_types=[
            pltpu.VMEM((NKP,), jnp.float32),
            pltpu.VMEM((BINS,), jnp.float32),
            pltpu.VMEM((CH2,), jnp.int32),
            pltpu.VMEM((CH2,), jnp.int32),
            pltpu.VMEM((CH2,), jnp.int32),
            pltpu.VMEM((CH2,), jnp.int32),
            pltpu.VMEM((CH2,), jnp.float32),
        ],
    )


_KB = 125
_CHR = 16


def _make_aggregate(N, E, R, D):
    K = _KB
    NW = NC * NS
    EPT = E // NW
    NB = EPT // K
    NCH = NB // _CHR
    EC = _CHR * K
    RPT = -(-(N // NS) // 8) * 8
    NP = NS * RPT
    assert EPT % K == 0 and NB % _CHR == 0 and _CHR % 2 == 0
    NJ = D // L
    NZ = RPT // K
    ZREM = RPT - NZ * K

    def body(y_hbm, gidx_hbm, dst_hbm, s_hbm, out_hbm,
             acc_sh, gidx_c0, gidx_c1, dst_c0, dst_c1, s_c0, s_c1,
             rows0, rows1, st0, st1, sg0, sg1, ss0, ss1):
        cid = lax.axis_index("c")
        sid = lax.axis_index("s")
        wid = cid * NS + sid
        st = (st0, st1)
        gi = (gidx_c0, gidx_c1)
        di = (dst_c0, dst_c1)
        sv = (s_c0, s_c1)

        def stage(ch, q):
            pltpu.async_copy(gidx_hbm.at[wid, pl.ds(ch * _CHR, _CHR)], gi[q], st[q])
            pltpu.async_copy(dst_hbm.at[wid, pl.ds(ch * _CHR, _CHR)], di[q], st[q])
            pltpu.async_copy(s_hbm.at[pl.ds(wid * EPT + ch * EC, EC)],
                             sv[q].at[pl.ds(0, EC)], st[q])

        def stage_wait(ch, q):
            pltpu.make_async_copy(gidx_hbm.at[wid, pl.ds(ch * _CHR, _CHR)],
                                  gi[q], st[q]).wait()
            pltpu.make_async_copy(dst_hbm.at[wid, pl.ds(ch * _CHR, _CHR)],
                                  di[q], st[q]).wait()
            pltpu.make_async_copy(s_hbm.at[pl.ds(wid * EPT + ch * EC, EC)],
                                  sv[q].at[pl.ds(0, EC)], st[q]).wait()

        stage(0, 0)
        if NCH > 1:
            stage(1, 1)

        def zfill(i, _):
            for j in range(NJ):
                rows0[i, pl.ds(j * L, L)] = jnp.zeros((L,), jnp.float32)
            return 0
        lax.fori_loop(0, K, zfill, 0)

        def zcopy(j, _):
            pltpu.sync_copy(rows0, acc_sh.at[pl.ds(sid * RPT + j * K, K)])
            return 0
        lax.fori_loop(0, NZ, zcopy, 0)
        if ZREM:
            pltpu.sync_copy(rows0.at[pl.ds(0, ZREM)],
                            acc_sh.at[pl.ds(sid * RPT + NZ * K, ZREM)])
        plsc.subcore_barrier()

        def gather(q, b, buf, sem):
            pltpu.async_copy(y_hbm.at[gi[q].at[b]], buf, sem)

        def gather_wait(q, b, buf, sem):
            pltpu.make_async_copy(y_hbm.at[gi[q].at[b]], buf, sem).wait()

        def scatter(q, b, buf, sem):
            pltpu.async_copy(buf, acc_sh.at[di[q].at[b]], sem, add=True)

        def scatter_wait(q, b, buf, sem):
            pltpu.make_async_copy(buf, acc_sh.at[di[q].at[b]], sem).wait()

        def scale(q, b, buf):
            def scale_e(i, _):
                e = 2 * i
                se0 = sv[q][pl.ds(b * K + e, L)][0]
                se1 = sv[q][pl.ds(b * K + e + 1, L)][0]
                for j in range(NJ):
                    buf[e, pl.ds(j * L, L)] = buf[e, pl.ds(j * L, L)] * se0
                for j in range(NJ):
                    buf[e + 1, pl.ds(j * L, L)] = buf[e + 1, pl.ds(j * L, L)] * se1
                return 0
            lax.fori_loop(0, K // 2, scale_e, 0)
            if K % 2:
                se = sv[q][pl.ds(b * K + K - 1, L)][0]
                for j in range(NJ):
                    buf[K - 1, pl.ds(j * L, L)] = buf[K - 1, pl.ds(j * L, L)] * se

        def pair(q, b0, prefetch):
            b1 = b0 + 1
            gather_wait(q, b0, rows0, sg0)
            scale(q, b0, rows0)
            scatter(q, b0, rows0, ss0)
            gather_wait(q, b1, rows1, sg1)
            scale(q, b1, rows1)
            scatter(q, b1, rows1, ss1)
            scatter_wait(q, b0, rows0, ss0)
            if prefetch:
                gather(q, b0 + 2, rows0, sg0)
            scatter_wait(q, b1, rows1, ss1)
            if prefetch:
                gather(q, b1 + 2, rows1, sg1)
            return 0

        stage_wait(0, 0)
        gather(0, 0, rows0, sg0)
        gather(0, 1, rows1, sg1)
        for ch in range(NCH):
            q = ch % 2
            lax.fori_loop(0, _CHR // 2 - 1, lambda p, c: pair(q, 2 * p, True), 0)
            pair(q, _CHR - 2, False)
            if ch + 1 < NCH:
                stage_wait(ch + 1, 1 - q)
                gather(1 - q, 0, rows0, sg0)
                gather(1 - q, 1, rows1, sg1)
            if ch + 2 < NCH:
                stage(ch + 2, q)
        plsc.subcore_barrier()

        pltpu.sync_copy(acc_sh.at[pl.ds(sid * RPT, RPT)],
                        out_hbm.at[cid, pl.ds(sid * RPT, RPT)])

    return pl.kernel(
        body,
        out_type=jax.ShapeDtypeStruct((NC, NP, D), jnp.float32),
        mesh=_MESH,
        compiler_params=pltpu.CompilerParams(needs_layout_passes=False),
        scratch_types=[
            pltpu.VMEM_SHARED((NP, D), jnp.float32),
            pltpu.VMEM((_CHR, K), jnp.int32),
            pltpu.VMEM((_CHR, K), jnp.int32),
            pltpu.VMEM((_CHR, K), jnp.int32),
            pltpu.VMEM((_CHR, K), jnp.int32),
            pltpu.VMEM((EC + L,), jnp.float32),
            pltpu.VMEM((EC + L,), jnp.float32),
            pltpu.VMEM((K, D), jnp.float32),
            pltpu.VMEM((K, D), jnp.float32),
            pltpu.SemaphoreType.DMA,
            pltpu.SemaphoreType.DMA,
            pltpu.SemaphoreType.DMA,
            pltpu.SemaphoreType.DMA,
            pltpu.SemaphoreType.DMA,
            pltpu.SemaphoreType.DMA,
        ],
    )


def _tc_first(x, wc, wr, BM=1000):
    N, D = x.shape
    RD = wc.shape[1]

    def body(x_ref, wc_ref, wr_ref, y_ref, z_ref):
        xb = x_ref[...]
        y_ref[...] = jnp.dot(xb, wc_ref[...], preferred_element_type=jnp.float32)
        z_ref[...] = jnp.dot(xb, wr_ref[...], preferred_element_type=jnp.float32)

    return pl.pallas_call(
        body,
        grid=(N // BM,),
        in_specs=[
            pl.BlockSpec((BM, D), lambda i: (i, 0)),
            pl.BlockSpec((D, RD), lambda i: (0, 0)),
            pl.BlockSpec((D, D), lambda i: (0, 0)),
        ],
        out_specs=[
            pl.BlockSpec((BM, RD), lambda i: (i, 0)),
            pl.BlockSpec((BM, D), lambda i: (i, 0)),
        ],
        out_shape=[
            jax.ShapeDtypeStruct((N, RD), jnp.float32),
            jax.ShapeDtypeStruct((N, D), jnp.float32),
        ],
    )(x, wc, wr)


def _tc_mid(z, a0, a1, b, wc, wr, BM=1000):
    N, D = z.shape
    RD = wc.shape[1]

    def body(z_ref, a0_ref, a1_ref, b_ref, wc_ref, wr_ref, y_ref, zo_ref):
        h = jnp.maximum(z_ref[...] + a0_ref[...] + a1_ref[...] + b_ref[...], 0.0)
        y_ref[...] = jnp.dot(h, wc_ref[...], preferred_element_type=jnp.float32)
        zo_ref[...] = jnp.dot(h, wr_ref[...], preferred_element_type=jnp.float32)

    return pl.pallas_call(
        body,
        grid=(N // BM,),
        in_specs=[
            pl.BlockSpec((BM, D), lambda i: (i, 0)),
            pl.BlockSpec((BM, D), lambda i: (i, 0)),
            pl.BlockSpec((BM, D), lambda i: (i, 0)),
            pl.BlockSpec((1, D), lambda i: (0, 0)),
            pl.BlockSpec((D, RD), lambda i: (0, 0)),
            pl.BlockSpec((D, D), lambda i: (0, 0)),
        ],
        out_specs=[
            pl.BlockSpec((BM, RD), lambda i: (i, 0)),
            pl.BlockSpec((BM, D), lambda i: (i, 0)),
        ],
        out_shape=[
            jax.ShapeDtypeStruct((N, RD), jnp.float32),
            jax.ShapeDtypeStruct((N, D), jnp.float32),
        ],
    )(z, a0, a1, b, wc, wr)


def _tc_last(z, a0, a1, b, BM=1000):
    N, D = z.shape

    def body(z_ref, a0_ref, a1_ref, b_ref, h_ref):
        h_ref[...] = jnp.maximum(
            z_ref[...] + a0_ref[...] + a1_ref[...] + b_ref[...], 0.0)

    return pl.pallas_call(
        body,
        grid=(N // BM,),
        in_specs=[
            pl.BlockSpec((BM, D), lambda i: (i, 0)),
            pl.BlockSpec((BM, D), lambda i: (i, 0)),
            pl.BlockSpec((BM, D), lambda i: (i, 0)),
            pl.BlockSpec((1, D), lambda i: (0, 0)),
        ],
        out_specs=pl.BlockSpec((BM, D), lambda i: (i, 0)),
        out_shape=jax.ShapeDtypeStruct((N, D), jnp.float32),
    )(z, a0, a1, b)


def kernel(x, edge_index, edge_type, w1, r1, b1, w2, r2, b2, w3, r3, b3):
    N, D = x.shape
    E = edge_type.shape[0]
    R = w1.shape[0]

    src = edge_index[0].astype(jnp.int32)
    dst = edge_index[1].astype(jnp.int32)
    typ = edge_type.astype(jnp.int32)

    cnta, cntb = _make_hist(N, E, R)(dst, typ)
    gidx, s = _make_edgeprep(N, E, R)(src, dst, typ, cnta, cntb)
    agg = _make_aggregate(N, E, R, D)
    NW = NC * NS
    gidx2 = gidx.reshape(NW, E // (NW * _KB), _KB)
    dst2 = dst.reshape(NW, E // (NW * _KB), _KB)

    def wcat(w):
        return jnp.transpose(w, (1, 0, 2)).reshape(D, R * D)

    y1, z1 = _tc_first(x, wcat(w1), r1)
    acc1 = agg(y1.reshape(N * R, D), gidx2, dst2, s)
    y2, z2 = _tc_mid(z1, acc1[0], acc1[1], b1.reshape(1, D), wcat(w2), r2)
    acc2 = agg(y2.reshape(N * R, D), gidx2, dst2, s)
    y3, z3 = _tc_mid(z2, acc2[0], acc2[1], b2.reshape(1, D), wcat(w3), r3)
    acc3 = agg(y3.reshape(N * R, D), gidx2, dst2, s)
    return _tc_last(z3, acc3[0], acc3[1], b3.reshape(1, D))


# speedup vs baseline: 1.4769x; 1.0137x over previous
import functools

import jax
import jax.numpy as jnp
from jax import lax
from jax.experimental import pallas as pl
from jax.experimental.pallas import tpu as pltpu
from jax.experimental.pallas import tpu_sc as plsc

NC = 2
NS = 16
L = 16

_MESH = plsc.VectorSubcoreMesh(
    core_axis_name="c", subcore_axis_name="s", num_cores=NC, num_subcores=NS)


def _bins_per_tile(N, R):
    return ((N * R + NS - 1) // NS + 15) // 16 * 16


def _make_hist(N, E, R):
    BINS = _bins_per_tile(N, R)
    NKP = NS * BINS
    CH = 4000
    EPC = E // NC
    assert EPC % CH == 0 and BINS % 8 == 0

    def body(dst_hbm, typ_hbm, outa_hbm, outb_hbm, cnt_part,
             st_a0, st_b0, st_a1, st_b1, sd0, sd1):
        cid = lax.axis_index("c")
        sid = lax.axis_index("s")
        lo = sid * BINS
        ones = jnp.ones((L,), jnp.float32)
        sts = ((st_a0, st_b0, sd0), (st_a1, st_b1, sd1))

        def stage(ch, q):
            a, bb, sem = sts[q]
            off = cid * EPC + ch * CH
            pltpu.async_copy(dst_hbm.at[pl.ds(off, CH)], a, sem)
            pltpu.async_copy(typ_hbm.at[pl.ds(off, CH)], bb, sem)

        def stage_wait(ch, q):
            a, bb, sem = sts[q]
            off = cid * EPC + ch * CH
            pltpu.make_async_copy(dst_hbm.at[pl.ds(off, CH)], a, sem).wait()
            pltpu.make_async_copy(typ_hbm.at[pl.ds(off, CH)], bb, sem).wait()

        stage(0, 0)
        stage(1, 1)

        @plsc.parallel_loop(0, BINS // L, 1, unroll=4)
        def zero_body(i):
            cnt_part[pl.ds(i * L, L)] = jnp.zeros((L,), jnp.float32)

        def count_chunk(ch, _):
            q = lax.rem(ch, 2)

            def scan(a, bb):
                @plsc.parallel_loop(0, CH // L, 1, unroll=4)
                def count_vec(i):
                    dd = a[pl.ds(i * L, L)]
                    tt = bb[pl.ds(i * L, L)]
                    k = dd * R + tt - lo
                    m = (k >= 0) & (k < BINS)
                    kk = jnp.where(m, k, 0)
                    plsc.addupdate_scatter(cnt_part, [kk], ones, mask=m)

            def do(q_static):
                a, bb, _ = sts[q_static]
                stage_wait(ch, q_static)
                scan(a, bb)

                def prefetch():
                    stage(ch + 2, q_static)
                lax.cond(ch + 2 < EPC // CH, prefetch, lambda: None)

            lax.cond(q == 0, lambda: do(0), lambda: do(1))
            return 0
        lax.fori_loop(0, EPC // CH, count_chunk, 0)

        lax.cond(
            cid == 0,
            lambda: pltpu.sync_copy(cnt_part, outa_hbm.at[pl.ds(lo, BINS)]),
            lambda: pltpu.sync_copy(cnt_part, outb_hbm.at[pl.ds(lo, BINS)]))

    return pl.kernel(
        body,
        out_type=(jax.ShapeDtypeStruct((NKP,), jnp.float32),
                  jax.ShapeDtypeStruct((NKP,), jnp.float32)),
        mesh=_MESH,
        compiler_params=pltpu.CompilerParams(needs_layout_passes=False),
        scratch_types=[
            pltpu.VMEM((BINS,), jnp.float32),
            pltpu.VMEM((CH,), jnp.int32),
            pltpu.VMEM((CH,), jnp.int32),
            pltpu.VMEM((CH,), jnp.int32),
            pltpu.VMEM((CH,), jnp.int32),
            pltpu.SemaphoreType.DMA,
            pltpu.SemaphoreType.DMA,
        ],
    )


def _make_edgeprep(N, E, R):
    BINS = _bins_per_tile(N, R)
    NKP = NS * BINS
    EPT = E // (NC * NS)
    CH2 = 2000
    assert EPT % CH2 == 0

    def body(src_hbm, dst_hbm, typ_hbm, cnta_hbm, cntb_hbm, gidx_hbm, s_hbm,
             cnt_full, cbuf, st_s, st_d, st_t, st_g, st_sc):
        cid = lax.axis_index("c")
        sid = lax.axis_index("s")
        wid = cid * NS + sid

        pltpu.sync_copy(cnta_hbm, cnt_full)

        def merge_chunk(ch, _):
            pltpu.sync_copy(cntb_hbm.at[pl.ds(ch * BINS, BINS)], cbuf)

            @plsc.parallel_loop(0, BINS // L, 1, unroll=4)
            def merge_vec(i):
                off = ch * BINS + i * L
                cnt_full[pl.ds(off, L)] = (cnt_full[pl.ds(off, L)]
                                           + cbuf[pl.ds(i * L, L)])
            return 0
        lax.fori_loop(0, NS, merge_chunk, 0)

        base = wid * EPT

        def out_chunk(ch, _):
            off = base + ch * CH2
            pltpu.sync_copy(src_hbm.at[pl.ds(off, CH2)], st_s)
            pltpu.sync_copy(dst_hbm.at[pl.ds(off, CH2)], st_d)
            pltpu.sync_copy(typ_hbm.at[pl.ds(off, CH2)], st_t)

            @plsc.parallel_loop(0, CH2 // L, 1, unroll=2)
            def out_vec(i):
                ss = st_s[pl.ds(i * L, L)]
                dd = st_d[pl.ds(i * L, L)]
                tt = st_t[pl.ds(i * L, L)]
                st_g[pl.ds(i * L, L)] = ss * R + tt
                c = plsc.load_gather(cnt_full, [dd * R + tt])
                st_sc[pl.ds(i * L, L)] = 1.0 / jnp.maximum(c, 1.0)

            pltpu.sync_copy(st_g, gidx_hbm.at[pl.ds(off, CH2)])
            pltpu.sync_copy(st_sc, s_hbm.at[pl.ds(off, CH2)])
            return 0
        lax.fori_loop(0, EPT // CH2, out_chunk, 0)

    return pl.kernel(
        body,
        out_type=(jax.ShapeDtypeStruct((E,), jnp.int32),
                  jax.ShapeDtypeStruct((E,), jnp.float32)),
        mesh=_MESH,
        compiler_params=pltpu.CompilerParams(needs_layout_passes=False),
        scratch_types=[
            pltpu.VMEM((NKP,), jnp.float32),
            pltpu.VMEM((BINS,), jnp.float32),
            pltpu.VMEM((CH2,), jnp.int32),
            pltpu.VMEM((CH2,), jnp.int32),
            pltpu.VMEM((CH2,), jnp.int32),
            pltpu.VMEM((CH2,), jnp.int32),
            pltpu.VMEM((CH2,), jnp.float32),
        ],
    )


_KB = 125
_CHR = 16


def _make_aggregate(N, E, R, D):
    K = _KB
    NW = NC * NS
    EPT = E // NW
    NB = EPT // K
    NCH = NB // _CHR
    EC = _CHR * K
    RPT = -(-(N // NS) // 8) * 8
    NP = NS * RPT
    assert EPT % K == 0 and NB % _CHR == 0 and _CHR % 2 == 0
    NJ = D // L
    NZ = RPT // K
    ZREM = RPT - NZ * K

    def body(y_hbm, gidx_hbm, dst_hbm, s_hbm, out_hbm,
             acc_sh, gidx_c0, gidx_c1, dst_c0, dst_c1, s_c0, s_c1,
             rows0, rows1, st0, st1, sg0, sg1, ss0, ss1):
        cid = lax.axis_index("c")
        sid = lax.axis_index("s")
        wid = cid * NS + sid
        st = (st0, st1)
        gi = (gidx_c0, gidx_c1)
        di = (dst_c0, dst_c1)
        sv = (s_c0, s_c1)

        def stage(ch, q):
            pltpu.async_copy(gidx_hbm.at[wid, pl.ds(ch * _CHR, _CHR)], gi[q], st[q])
            pltpu.async_copy(dst_hbm.at[wid, pl.ds(ch * _CHR, _CHR)], di[q], st[q])
            pltpu.async_copy(s_hbm.at[pl.ds(wid * EPT + ch * EC, EC)],
                             sv[q].at[pl.ds(0, EC)], st[q])

        def stage_wait(ch, q):
            pltpu.make_async_copy(gidx_hbm.at[wid, pl.ds(ch * _CHR, _CHR)],
                                  gi[q], st[q]).wait()
            pltpu.make_async_copy(dst_hbm.at[wid, pl.ds(ch * _CHR, _CHR)],
                                  di[q], st[q]).wait()
            pltpu.make_async_copy(s_hbm.at[pl.ds(wid * EPT + ch * EC, EC)],
                                  sv[q].at[pl.ds(0, EC)], st[q]).wait()

        stage(0, 0)
        if NCH > 1:
            stage(1, 1)

        def zfill(i, _):
            for j in range(NJ):
                rows0[i, pl.ds(j * L, L)] = jnp.zeros((L,), jnp.float32)
            return 0
        lax.fori_loop(0, K, zfill, 0)

        def zcopy(j, _):
            pltpu.sync_copy(rows0, acc_sh.at[pl.ds(sid * RPT + j * K, K)])
            return 0
        lax.fori_loop(0, NZ, zcopy, 0)
        if ZREM:
            pltpu.sync_copy(rows0.at[pl.ds(0, ZREM)],
                            acc_sh.at[pl.ds(sid * RPT + NZ * K, ZREM)])
        plsc.subcore_barrier()

        def gather(q, b, buf, sem):
            pltpu.async_copy(y_hbm.at[gi[q].at[b]], buf, sem)

        def gather_wait(q, b, buf, sem):
            pltpu.make_async_copy(y_hbm.at[gi[q].at[b]], buf, sem).wait()

        def scatter(q, b, buf, sem):
            pltpu.async_copy(buf, acc_sh.at[di[q].at[b]], sem, add=True)

        def scatter_wait(q, b, buf, sem):
            pltpu.make_async_copy(buf, acc_sh.at[di[q].at[b]], sem).wait()

        def scale(q, b, buf):
            def scale_e(i, _):
                e = 2 * i
                se0 = sv[q][pl.ds(b * K + e, L)][0]
                se1 = sv[q][pl.ds(b * K + e + 1, L)][0]
                for j in range(NJ):
                    buf[e, pl.ds(j * L, L)] = buf[e, pl.ds(j * L, L)] * se0
                for j in range(NJ):
                    buf[e + 1, pl.ds(j * L, L)] = buf[e + 1, pl.ds(j * L, L)] * se1
                return 0
            lax.fori_loop(0, K // 2, scale_e, 0)
            if K % 2:
                se = sv[q][pl.ds(b * K + K - 1, L)][0]
                for j in range(NJ):
                    buf[K - 1, pl.ds(j * L, L)] = buf[K - 1, pl.ds(j * L, L)] * se

        def pair(q, b0, prefetch):
            b1 = b0 + 1
            gather_wait(q, b0, rows0, sg0)
            scale(q, b0, rows0)
            scatter(q, b0, rows0, ss0)
            gather_wait(q, b1, rows1, sg1)
            scale(q, b1, rows1)
            scatter(q, b1, rows1, ss1)
            scatter_wait(q, b0, rows0, ss0)
            if prefetch:
                gather(q, b0 + 2, rows0, sg0)
            scatter_wait(q, b1, rows1, ss1)
            if prefetch:
                gather(q, b1 + 2, rows1, sg1)
            return 0

        stage_wait(0, 0)
        gather(0, 0, rows0, sg0)
        gather(0, 1, rows1, sg1)
        for ch in range(NCH):
            q = ch % 2
            lax.fori_loop(0, _CHR // 2 - 1, lambda p, c: pair(q, 2 * p, True), 0)
            pair(q, _CHR - 2, False)
            if ch + 1 < NCH:
                stage_wait(ch + 1, 1 - q)
                gather(1 - q, 0, rows0, sg0)
                gather(1 - q, 1, rows1, sg1)
            if ch + 2 < NCH:
                stage(ch + 2, q)
        plsc.subcore_barrier()

        pltpu.sync_copy(acc_sh.at[pl.ds(sid * RPT, RPT)],
                        out_hbm.at[cid, pl.ds(sid * RPT, RPT)])

    return pl.kernel(
        body,
        out_type=jax.ShapeDtypeStruct((NC, NP, D), jnp.float32),
        mesh=_MESH,
        compiler_params=pltpu.CompilerParams(needs_layout_passes=False),
        scratch_types=[
            pltpu.VMEM_SHARED((NP, D), jnp.float32),
            pltpu.VMEM((_CHR, K), jnp.int32),
            pltpu.VMEM((_CHR, K), jnp.int32),
            pltpu.VMEM((_CHR, K), jnp.int32),
            pltpu.VMEM((_CHR, K), jnp.int32),
            pltpu.VMEM((EC + L,), jnp.float32),
            pltpu.VMEM((EC + L,), jnp.float32),
            pltpu.VMEM((K, D), jnp.float32),
            pltpu.VMEM((K, D), jnp.float32),
            pltpu.SemaphoreType.DMA,
            pltpu.SemaphoreType.DMA,
            pltpu.SemaphoreType.DMA,
            pltpu.SemaphoreType.DMA,
            pltpu.SemaphoreType.DMA,
            pltpu.SemaphoreType.DMA,
        ],
    )


def _tc_first(x, wc, wr, BM=2000):
    N, D = x.shape
    RD = wc.shape[1]

    def body(x_ref, wc_ref, wr_ref, y_ref, z_ref):
        xb = x_ref[...]
        y_ref[...] = jnp.dot(xb, wc_ref[...], preferred_element_type=jnp.float32)
        z_ref[...] = jnp.dot(xb, wr_ref[...], preferred_element_type=jnp.float32)

    return pl.pallas_call(
        body,
        grid=(N // BM,),
        in_specs=[
            pl.BlockSpec((BM, D), lambda i: (i, 0)),
            pl.BlockSpec((D, RD), lambda i: (0, 0)),
            pl.BlockSpec((D, D), lambda i: (0, 0)),
        ],
        out_specs=[
            pl.BlockSpec((BM, RD), lambda i: (i, 0)),
            pl.BlockSpec((BM, D), lambda i: (i, 0)),
        ],
        out_shape=[
            jax.ShapeDtypeStruct((N, RD), jnp.float32),
            jax.ShapeDtypeStruct((N, D), jnp.float32),
        ],
    )(x, wc, wr)


def _tc_mid(z, a0, a1, b, wc, wr, BM=2000):
    N, D = z.shape
    RD = wc.shape[1]

    def body(z_ref, a0_ref, a1_ref, b_ref, wc_ref, wr_ref, y_ref, zo_ref):
        h = jnp.maximum(z_ref[...] + a0_ref[...] + a1_ref[...] + b_ref[...], 0.0)
        y_ref[...] = jnp.dot(h, wc_ref[...], preferred_element_type=jnp.float32)
        zo_ref[...] = jnp.dot(h, wr_ref[...], preferred_element_type=jnp.float32)

    return pl.pallas_call(
        body,
        grid=(N // BM,),
        in_specs=[
            pl.BlockSpec((BM, D), lambda i: (i, 0)),
            pl.BlockSpec((BM, D), lambda i: (i, 0)),
            pl.BlockSpec((BM, D), lambda i: (i, 0)),
            pl.BlockSpec((1, D), lambda i: (0, 0)),
            pl.BlockSpec((D, RD), lambda i: (0, 0)),
            pl.BlockSpec((D, D), lambda i: (0, 0)),
        ],
        out_specs=[
            pl.BlockSpec((BM, RD), lambda i: (i, 0)),
            pl.BlockSpec((BM, D), lambda i: (i, 0)),
        ],
        out_shape=[
            jax.ShapeDtypeStruct((N, RD), jnp.float32),
            jax.ShapeDtypeStruct((N, D), jnp.float32),
        ],
    )(z, a0, a1, b, wc, wr)


def _tc_last(z, a0, a1, b, BM=2000):
    N, D = z.shape

    def body(z_ref, a0_ref, a1_ref, b_ref, h_ref):
        h_ref[...] = jnp.maximum(
            z_ref[...] + a0_ref[...] + a1_ref[...] + b_ref[...], 0.0)

    return pl.pallas_call(
        body,
        grid=(N // BM,),
        in_specs=[
            pl.BlockSpec((BM, D), lambda i: (i, 0)),
            pl.BlockSpec((BM, D), lambda i: (i, 0)),
            pl.BlockSpec((BM, D), lambda i: (i, 0)),
            pl.BlockSpec((1, D), lambda i: (0, 0)),
        ],
        out_specs=pl.BlockSpec((BM, D), lambda i: (i, 0)),
        out_shape=jax.ShapeDtypeStruct((N, D), jnp.float32),
    )(z, a0, a1, b)


def kernel(x, edge_index, edge_type, w1, r1, b1, w2, r2, b2, w3, r3, b3):
    N, D = x.shape
    E = edge_type.shape[0]
    R = w1.shape[0]

    src = edge_index[0].astype(jnp.int32)
    dst = edge_index[1].astype(jnp.int32)
    typ = edge_type.astype(jnp.int32)

    cnta, cntb = _make_hist(N, E, R)(dst, typ)
    gidx, s = _make_edgeprep(N, E, R)(src, dst, typ, cnta, cntb)
    agg = _make_aggregate(N, E, R, D)
    NW = NC * NS
    gidx2 = gidx.reshape(NW, E // (NW * _KB), _KB)
    dst2 = dst.reshape(NW, E // (NW * _KB), _KB)

    def wcat(w):
        return jnp.transpose(w, (1, 0, 2)).reshape(D, R * D)

    y1, z1 = _tc_first(x, wcat(w1), r1)
    acc1 = agg(y1.reshape(N * R, D), gidx2, dst2, s)
    y2, z2 = _tc_mid(z1, acc1[0], acc1[1], b1.reshape(1, D), wcat(w2), r2)
    acc2 = agg(y2.reshape(N * R, D), gidx2, dst2, s)
    y3, z3 = _tc_mid(z2, acc2[0], acc2[1], b2.reshape(1, D), wcat(w3), r3)
    acc3 = agg(y3.reshape(N * R, D), gidx2, dst2, s)
    return _tc_last(z3, acc3[0], acc3[1], b3.reshape(1, D))
